# baseline (speedup 1.0000x reference)
import functools

import jax
import jax.numpy as jnp
from jax import lax
from jax.experimental import pallas as pl
from jax.experimental.pallas import tpu as pltpu
from jax.experimental.pallas import tpu_sc as plsc
from jax._src.pallas import mpmd as _mpmd

N_EDGES = 4_000_000
N_SEL = 400_000

_G = 50
_R = 625
_L = 128

_NC, _NS = 2, 16
_NW = _NC * _NS
_ROW = 128
_ROWS_W = 104
_SEL_PAD = _NW * _ROWS_W * _ROW


def _dense_body(m_ref, a_ref, o_ref):
    o_ref[...] = a_ref[...] * jax.nn.sigmoid(m_ref[...])


_dense = pl.pallas_call(
    _dense_body,
    grid=(_G,),
    in_specs=[
        pl.BlockSpec((1, _R, _L), lambda i: (i, 0, 0)),
        pl.BlockSpec((1, _R, _L), lambda i: (i, 0, 0)),
    ],
    out_specs=pl.BlockSpec((1, _R, _L), lambda i: (i, 0, 0)),
    out_shape=jax.ShapeDtypeStruct((_G, _R, _L), jnp.float32),
)


def _fix_body(out_in, idx_hbm, adj_hbm, out_hbm, idx_v, vals_v, sem):
    del out_in
    wid = lax.axis_index("s") * _NC + lax.axis_index("c")
    row0 = wid * _ROWS_W
    pltpu.sync_copy(idx_hbm.at[pl.ds(row0, _ROWS_W)], idx_v)
    gathers = [
        pltpu.async_copy(
            adj_hbm.at[idx_v.at[j]], vals_v.at[pl.ds(j * _ROW, _ROW)], sem
        )
        for j in range(_ROWS_W)
    ]
    for g in gathers:
        g.wait()

    def _scale(i, carry):
        s = pl.multiple_of(i * 16, 16)
        vals_v[pl.ds(s, 16)] = vals_v[pl.ds(s, 16)] * 0.5
        return carry

    lax.fori_loop(0, _ROWS_W * _ROW // 16, _scale, 0)
    scatters = [
        pltpu.async_copy(
            vals_v.at[pl.ds(j * _ROW, _ROW)], out_hbm.at[idx_v.at[j]], sem
        )
        for j in range(_ROWS_W)
    ]
    for s in scatters:
        s.wait()


@functools.cache
def _get_fix():
    mesh = plsc.VectorSubcoreMesh(
        core_axis_name="c", subcore_axis_name="s",
        num_cores=_NC, num_subcores=_NS,
    )
    return _mpmd._mpmd_map(
        [(mesh, _fix_body)],
        jax.ShapeDtypeStruct((N_EDGES,), jnp.float32),
        input_output_aliases={0: 0},
        scratch_types=[
            pltpu.VMEM((_ROWS_W, _ROW), jnp.int32),
            pltpu.VMEM((_ROWS_W * _ROW,), jnp.float32),
            pltpu.SemaphoreType.DMA,
        ],
    )


def kernel(mask, idx, adj_values):
    mask3 = mask.reshape(_G, _R, _L)
    adj3 = adj_values.reshape(_G, _R, _L)
    out0 = _dense(mask3, adj3).reshape(N_EDGES)
    idx32 = idx.astype(jnp.int32)
    idx_pad = jnp.concatenate(
        [idx32, jnp.broadcast_to(idx32[0], (_SEL_PAD - N_SEL,))]
    ).reshape(_SEL_PAD // _ROW, _ROW)
    return _get_fix()(out0, idx_pad, adj_values)


# speedup vs baseline: 8.1531x; 8.1531x over previous
import functools

import jax
import jax.numpy as jnp
from jax import lax
from jax.experimental import pallas as pl
from jax.experimental.pallas import tpu as pltpu
from jax.experimental.pallas import tpu_sc as plsc
from jax._src.pallas import mpmd as _mpmd

N_EDGES = 4_000_000
N_SEL = 400_000

_G = 50
_R = 625
_L = 128

_NC, _NS = 2, 16
_NW = _NC * _NS
_T = 12_512
_SEL_PAD = _NW * _T


def _dense_body(m_ref, a_ref, o_ref):
    o_ref[...] = a_ref[...] * jax.nn.sigmoid(m_ref[...])


_dense = pl.pallas_call(
    _dense_body,
    grid=(_G,),
    in_specs=[
        pl.BlockSpec((1, _R, _L), lambda i: (i, 0, 0)),
        pl.BlockSpec((1, _R, _L), lambda i: (i, 0, 0)),
    ],
    out_specs=pl.BlockSpec((1, _R, _L), lambda i: (i, 0, 0)),
    out_shape=jax.ShapeDtypeStruct((_G, _R, _L), jnp.float32),
)


def _fix_body(out_in, idx_hbm, adj_hbm, out_hbm, idx_v, vals_v, sem):
    del out_in
    wid = lax.axis_index("s") * _NC + lax.axis_index("c")
    base = wid * _T
    pltpu.sync_copy(idx_hbm.at[pl.ds(base, _T)], idx_v)
    pltpu.async_copy(adj_hbm.at[idx_v], vals_v, sem).wait()

    def _scale(i, carry):
        s = pl.multiple_of(i * 16, 16)
        vals_v[pl.ds(s, 16)] = vals_v[pl.ds(s, 16)] * 0.5
        return carry

    lax.fori_loop(0, _T // 16, _scale, 0)
    pltpu.async_copy(vals_v, out_hbm.at[idx_v], sem).wait()


@functools.cache
def _get_fix():
    mesh = plsc.VectorSubcoreMesh(
        core_axis_name="c", subcore_axis_name="s",
        num_cores=_NC, num_subcores=_NS,
    )
    return _mpmd._mpmd_map(
        [(mesh, _fix_body)],
        jax.ShapeDtypeStruct((N_EDGES,), jnp.float32),
        input_output_aliases={0: 0},
        scratch_types=[
            pltpu.VMEM((_T,), jnp.int32),
            pltpu.VMEM((_T,), jnp.float32),
            pltpu.SemaphoreType.DMA,
        ],
    )


def kernel(mask, idx, adj_values):
    mask3 = mask.reshape(_G, _R, _L)
    adj3 = adj_values.reshape(_G, _R, _L)
    out0 = _dense(mask3, adj3).reshape(N_EDGES)
    idx32 = idx.astype(jnp.int32)
    idx_pad = jnp.concatenate(
        [idx32, jnp.broadcast_to(idx32[0], (_SEL_PAD - N_SEL,))]
    )
    return _get_fix()(out0, idx_pad, adj_values)
